# baseline (speedup 1.0000x reference)
import functools

import jax
import jax.numpy as jnp
from jax import lax
from jax.experimental import pallas as pl
from jax.experimental.pallas import tpu as pltpu
from jax.experimental.pallas import tpu_sc as plsc

NC = 2
NS = 16
NW = NC * NS
LANES = 16
CB = 1024


def _mesh():
  return plsc.VectorSubcoreMesh(core_axis_name="c", subcore_axis_name="s",
                                num_cores=NC, num_subcores=NS)


def _zero_flat(acc_v, nwords):
  z = jnp.zeros((LANES,), jnp.float32)
  def zb(i, _):
    acc_v[pl.ds(i * LANES, LANES)] = z
    return 0
  lax.fori_loop(0, nwords // LANES, zb, 0)


def _make_deg(NP, EP, TT):
  EC = EP // NW
  CBD = 640
  assert EC % CBD == 0 and CBD % 8 == 0
  NCH = EC // CBD

  @functools.partial(
      pl.kernel,
      out_type=jax.ShapeDtypeStruct((NW * TT * NP,), jnp.float32),
      mesh=_mesh(),
      compiler_params=pltpu.CompilerParams(needs_layout_passes=False),
      scratch_types=[
          pltpu.VMEM((TT * NP,), jnp.float32),
          pltpu.VMEM((CBD,), jnp.int32),
          pltpu.VMEM((TT * CBD,), jnp.float32),
      ],
  )
  def deg_kernel(row_hbm, w_hbm, out_hbm, acc_v, row_cb, w_cb):
    c = lax.axis_index("c")
    s = lax.axis_index("s")
    wid = c * NS + s
    ebase = wid * EC
    _zero_flat(acc_v, TT * NP)
    iota = lax.iota(jnp.int32, 16)
    c4 = iota // 4
    trep = iota % 4
    tcb = trep * CBD
    tnp = trep * NP

    def chunk(j, _):
      pltpu.sync_copy(row_hbm.at[pl.ds(ebase + j * CBD, CBD)], row_cb)
      for t in range(TT):
        pltpu.sync_copy(w_hbm.at[pl.ds(t * EP + ebase + j * CBD, CBD)],
                        w_cb.at[pl.ds(t * CBD, CBD)])

      def group(g, _):
        erep = jnp.broadcast_to(g * 4, (16,)) + c4
        rrep = plsc.load_gather(row_cb, [erep])
        wv = plsc.load_gather(w_cb, [tcb + erep])
        plsc.addupdate_scatter(acc_v, [tnp + rrep], wv)
        return 0

      lax.fori_loop(0, CBD // 4, group, 0)
      return 0

    lax.fori_loop(0, NCH, chunk, 0)
    pltpu.sync_copy(acc_v, out_hbm.at[pl.ds(wid * TT * NP, TT * NP)])

  return deg_kernel


def _make_spmm(NP, EP, D, TT):
  LPW = D // NW
  NCH = EP // CB

  @functools.partial(
      pl.kernel,
      out_type=jax.ShapeDtypeStruct((TT * D * NP,), jnp.float32),
      mesh=_mesh(),
      compiler_params=pltpu.CompilerParams(needs_layout_passes=False),
      scratch_types=[
          pltpu.VMEM((LPW * NP,), jnp.float32),
          pltpu.VMEM((LPW * NP,), jnp.float32),
          pltpu.VMEM((NP,), jnp.float32),
          pltpu.VMEM((CB,), jnp.int32),
          pltpu.VMEM((CB,), jnp.int32),
          pltpu.VMEM((CB,), jnp.float32),
      ],
  )
  def spmm_kernel(xT_hbm, row_hbm, col_hbm, w_hbm, dis_hbm, out_hbm,
                  acc_v, xT_v, dis_v, row_cb, col_cb, w_cb):
    c = lax.axis_index("c")
    s = lax.axis_index("s")
    wid = c * NS + s
    iota = lax.iota(jnp.int32, 16)
    c4 = iota // 4
    lnp = (iota % 4) * NP

    for t in range(TT):
      slab = (t * D + LPW * wid) * NP
      pltpu.sync_copy(xT_hbm.at[pl.ds(slab, LPW * NP)], xT_v)
      pltpu.sync_copy(dis_hbm.at[pl.ds(t * NP, NP)], dis_v)
      _zero_flat(acc_v, LPW * NP)

      def chunk(j, _):
        pltpu.sync_copy(row_hbm.at[pl.ds(j * CB, CB)], row_cb)
        pltpu.sync_copy(col_hbm.at[pl.ds(j * CB, CB)], col_cb)
        pltpu.sync_copy(w_hbm.at[pl.ds(t * EP + j * CB, CB)], w_cb)

        def group(g, _):
          base = g * 32
          for u in range(8):
            erep = jnp.broadcast_to(base + u * 4, (16,)) + c4
            rrep = plsc.load_gather(row_cb, [erep])
            crep = plsc.load_gather(col_cb, [erep])
            wrep = plsc.load_gather(w_cb, [erep])
            drep = plsc.load_gather(dis_v, [rrep])
            vals = plsc.load_gather(xT_v, [lnp + rrep])
            msg = vals * (wrep * drep)
            plsc.addupdate_scatter(acc_v, [lnp + crep], msg)
          return 0

        lax.fori_loop(0, CB // 32, group, 0)
        return 0

      lax.fori_loop(0, NCH, chunk, 0)
      pltpu.sync_copy(acc_v, out_hbm.at[pl.ds(slab, LPW * NP)])

  return spmm_kernel


def _prep_body(degp_ref, dis_ref):
  deg = jnp.sum(degp_ref[...], axis=0)
  dis_ref[...] = jnp.where(deg > 0, lax.rsqrt(jnp.maximum(deg, 1e-30)), 0.0)


def _step_body(x_ref, lx_ref, lh_ref, h_ref, c_ref, dis_ref, w4_ref, b_ref,
               wc_ref, hn_ref, cn_ref):
  disr = dis_ref[...]
  lx = lx_ref[...] * (-disr)
  lh = lh_ref[...] * (-disr)
  h = h_ref[...]
  cc = c_ref[...]
  z = (jnp.dot(x_ref[...], w4_ref[0], preferred_element_type=jnp.float32)
       + jnp.dot(lx, w4_ref[1], preferred_element_type=jnp.float32)
       + jnp.dot(h, w4_ref[2], preferred_element_type=jnp.float32)
       + jnp.dot(lh, w4_ref[3], preferred_element_type=jnp.float32)
       + b_ref[...])
  dh = h.shape[1]
  zi, zf, zc, zo = (z[:, 0:dh], z[:, dh:2 * dh], z[:, 2 * dh:3 * dh],
                    z[:, 3 * dh:4 * dh])
  gi = jax.nn.sigmoid(zi + wc_ref[0:1] * cc)
  gf = jax.nn.sigmoid(zf + wc_ref[1:2] * cc)
  cn = gf * cc + gi * jnp.tanh(zc)
  go = jax.nn.sigmoid(zo + wc_ref[2:3] * cn)
  hn = go * jnp.tanh(cn)
  cn_ref[...] = cn
  hn_ref[...] = hn


def _pool_body(h_ref, b_ref, clsw_ref, clsb_ref, out_ref, *, G):
  n = h_ref.shape[0]
  gids = lax.broadcasted_iota(jnp.int32, (n, G), 1)
  onehot = (b_ref[...] == gids).astype(jnp.float32)
  sums = lax.dot_general(onehot, h_ref[...], (((0,), (0,)), ((), ())),
                         preferred_element_type=jnp.float32)
  cnt = jnp.sum(onehot, axis=0)
  pooled = sums / jnp.maximum(cnt, 1.0)[:, None]
  out_ref[...] = (jnp.dot(pooled, clsw_ref[...],
                          preferred_element_type=jnp.float32) + clsb_ref[...])


def kernel(x, edge_index, edge_attr, batch, conv_x_W, conv_x_b, conv_h_W,
           conv_h_b, w_c, b_gate, cls_W, cls_b):
  T, N, D = x.shape
  E = edge_index.shape[1]
  G = 16
  DH = conv_x_W.shape[-1]
  DO = cls_W.shape[1]
  EPU = NW * CB
  EP = ((E + EPU - 1) // EPU) * EPU
  NP = ((N + 127) // 128) * 128

  epad = EP - E
  row = jnp.pad(edge_index[0], (0, epad))
  col = jnp.pad(edge_index[1], (0, epad))
  w_flat = jnp.pad(edge_attr, ((0, 0), (0, epad))).reshape(-1)

  degp = _make_deg(NP, EP, T)(row, w_flat)

  dis = pl.pallas_call(
      _prep_body,
      out_shape=jax.ShapeDtypeStruct((T, NP), jnp.float32),
  )(degp.reshape(NW, T, NP))
  dis_flat = dis.reshape(-1)

  xT = jnp.pad(x.transpose(0, 2, 1), ((0, 0), (0, 0), (0, NP - N)))
  lxT = _make_spmm(NP, EP, D, T)(xT.reshape(-1), row, col, w_flat, dis_flat)
  lx = lxT.reshape(T, D, NP).transpose(0, 2, 1)
  disC = dis.T

  spmm1 = _make_spmm(NP, EP, D, 1)

  wx0 = jnp.transpose(conv_x_W[:, 0], (1, 0, 2)).reshape(D, 4 * DH)
  wx1 = jnp.transpose(conv_x_W[:, 1], (1, 0, 2)).reshape(D, 4 * DH)
  wh0 = jnp.transpose(conv_h_W[:, 0], (1, 0, 2)).reshape(DH, 4 * DH)
  wh1 = jnp.transpose(conv_h_W[:, 1], (1, 0, 2)).reshape(DH, 4 * DH)
  w4 = jnp.stack([wx0, wx1, wh0, wh1])
  bias = (conv_x_b + conv_h_b + b_gate).reshape(1, 4 * DH)

  R = 2000
  step_call = pl.pallas_call(
      _step_body,
      grid=(N // R,),
      in_specs=[
          pl.BlockSpec((R, D), lambda i: (i, 0)),
          pl.BlockSpec((R, D), lambda i: (i, 0)),
          pl.BlockSpec((R, DH), lambda i: (i, 0)),
          pl.BlockSpec((R, DH), lambda i: (i, 0)),
          pl.BlockSpec((R, DH), lambda i: (i, 0)),
          pl.BlockSpec((R, 1), lambda i: (i, 0)),
          pl.BlockSpec((4, D, 4 * DH), lambda i: (0, 0, 0)),
          pl.BlockSpec((1, 4 * DH), lambda i: (0, 0)),
          pl.BlockSpec((3, DH), lambda i: (0, 0)),
      ],
      out_specs=[
          pl.BlockSpec((R, DH), lambda i: (i, 0)),
          pl.BlockSpec((R, DH), lambda i: (i, 0)),
      ],
      out_shape=[
          jax.ShapeDtypeStruct((N, DH), jnp.float32),
          jax.ShapeDtypeStruct((N, DH), jnp.float32),
      ],
  )

  H = jnp.zeros((N, DH), jnp.float32)
  C = jnp.zeros((N, DH), jnp.float32)
  zero_lh = jnp.zeros((NP, DH), jnp.float32)
  for t in range(T):
    if t == 0:
      lh = zero_lh
    else:
      ht = jnp.pad(H.T, ((0, 0), (0, NP - N)))
      lh = spmm1(ht.reshape(-1), row, col, w_flat[t * EP:(t + 1) * EP],
                 dis_flat[t * NP:(t + 1) * NP]).reshape(DH, NP).T
    H, C = step_call(x[t], lx[t], lh, H, C, disC[:, t:t + 1], w4, bias, w_c)

  out = pl.pallas_call(
      functools.partial(_pool_body, G=G),
      out_shape=jax.ShapeDtypeStruct((G, DO), jnp.float32),
  )(H, batch.reshape(N, 1), cls_W, cls_b.reshape(1, DO))
  return out


# speedup vs baseline: 1.1825x; 1.1825x over previous
import functools

import jax
import jax.numpy as jnp
from jax import lax
from jax.experimental import pallas as pl
from jax.experimental.pallas import tpu as pltpu
from jax.experimental.pallas import tpu_sc as plsc

NC = 2
NS = 16
NW = NC * NS
LANES = 16
CB = 1024


def _mesh():
  return plsc.VectorSubcoreMesh(core_axis_name="c", subcore_axis_name="s",
                                num_cores=NC, num_subcores=NS)


def _zero_flat(acc_v, nwords):
  z = jnp.zeros((LANES,), jnp.float32)
  def zb(i, _):
    acc_v[pl.ds(i * LANES, LANES)] = z
    return 0
  lax.fori_loop(0, nwords // LANES, zb, 0)


def _make_deg(NP, EP, TT):
  EC = EP // NW
  CBD = 640
  assert EC % CBD == 0 and CBD % 8 == 0
  NCH = EC // CBD

  @functools.partial(
      pl.kernel,
      out_type=jax.ShapeDtypeStruct((NW * TT * NP,), jnp.float32),
      mesh=_mesh(),
      compiler_params=pltpu.CompilerParams(needs_layout_passes=False),
      scratch_types=[
          pltpu.VMEM((TT * NP,), jnp.float32),
          pltpu.VMEM((CBD,), jnp.int32),
          pltpu.VMEM((TT * CBD,), jnp.float32),
      ],
  )
  def deg_kernel(row_hbm, w_hbm, out_hbm, acc_v, row_cb, w_cb):
    c = lax.axis_index("c")
    s = lax.axis_index("s")
    wid = c * NS + s
    ebase = wid * EC
    _zero_flat(acc_v, TT * NP)
    iota = lax.iota(jnp.int32, 16)
    c4 = iota // 4
    trep = iota % 4
    tcb = trep * CBD
    tnp = trep * NP

    def chunk(j, _):
      pltpu.sync_copy(row_hbm.at[pl.ds(ebase + j * CBD, CBD)], row_cb)
      for t in range(TT):
        pltpu.sync_copy(w_hbm.at[pl.ds(t * EP + ebase + j * CBD, CBD)],
                        w_cb.at[pl.ds(t * CBD, CBD)])

      def group(g, _):
        erep = jnp.broadcast_to(g * 4, (16,)) + c4
        rrep = plsc.load_gather(row_cb, [erep])
        wv = plsc.load_gather(w_cb, [tcb + erep])
        plsc.addupdate_scatter(acc_v, [tnp + rrep], wv)
        return 0

      lax.fori_loop(0, CBD // 4, group, 0)
      return 0

    lax.fori_loop(0, NCH, chunk, 0)
    pltpu.sync_copy(acc_v, out_hbm.at[pl.ds(wid * TT * NP, TT * NP)])

  return deg_kernel


def _make_spmm(NP, EP, D, TT):
  LPW = D // NW
  NCH = EP // CB

  @functools.partial(
      pl.kernel,
      out_type=jax.ShapeDtypeStruct((TT * D * NP,), jnp.float32),
      mesh=_mesh(),
      compiler_params=pltpu.CompilerParams(needs_layout_passes=False),
      scratch_types=[
          pltpu.VMEM((LPW * NP,), jnp.float32),
          pltpu.VMEM((LPW * NP,), jnp.float32),
          pltpu.VMEM((NP,), jnp.float32),
          pltpu.VMEM((CB,), jnp.int32),
          pltpu.VMEM((CB,), jnp.int32),
          pltpu.VMEM((CB,), jnp.float32),
      ],
  )
  def spmm_kernel(xT_hbm, row_hbm, col_hbm, w_hbm, dis_hbm, out_hbm,
                  acc_v, xT_v, dis_v, row_cb, col_cb, w_cb):
    c = lax.axis_index("c")
    s = lax.axis_index("s")
    wid = c * NS + s
    iota = lax.iota(jnp.int32, 16)
    c4 = iota // 4
    lnp = (iota % 4) * NP

    for t in range(TT):
      slab = (t * D + LPW * wid) * NP
      pltpu.sync_copy(xT_hbm.at[pl.ds(slab, LPW * NP)], xT_v)
      pltpu.sync_copy(dis_hbm.at[pl.ds(t * NP, NP)], dis_v)
      _zero_flat(acc_v, LPW * NP)

      def chunk(j, _):
        pltpu.sync_copy(row_hbm.at[pl.ds(j * CB, CB)], row_cb)
        pltpu.sync_copy(col_hbm.at[pl.ds(j * CB, CB)], col_cb)
        pltpu.sync_copy(w_hbm.at[pl.ds(t * EP + j * CB, CB)], w_cb)

        def group(g, _):
          base = g * 32
          for u in range(8):
            erep = jnp.broadcast_to(base + u * 4, (16,)) + c4
            rrep = plsc.load_gather(row_cb, [erep])
            crep = plsc.load_gather(col_cb, [erep])
            wrep = plsc.load_gather(w_cb, [erep])
            drep = plsc.load_gather(dis_v, [rrep])
            vals = plsc.load_gather(xT_v, [lnp + rrep])
            msg = vals * (wrep * drep)
            plsc.addupdate_scatter(acc_v, [lnp + crep], msg)
          return 0

        lax.fori_loop(0, CB // 32, group, 0)
        return 0

      lax.fori_loop(0, NCH, chunk, 0)
      pltpu.sync_copy(acc_v, out_hbm.at[pl.ds(slab, LPW * NP)])

  return spmm_kernel


def _prep_body(degp_ref, dis_ref):
  deg = jnp.sum(degp_ref[...], axis=0)
  dis_ref[...] = jnp.where(deg > 0, lax.rsqrt(jnp.maximum(deg, 1e-30)), 0.0)


def _step_body(x_ref, lx_ref, lh_ref, h_ref, c_ref, dis_ref, w4_ref, b_ref,
               wc_ref, hn_ref, cn_ref):
  disr = dis_ref[...]
  lx = lx_ref[...] * (-disr)
  lh = lh_ref[...] * (-disr)
  h = h_ref[...]
  cc = c_ref[...]
  z = (jnp.dot(x_ref[...], w4_ref[0], preferred_element_type=jnp.float32)
       + jnp.dot(lx, w4_ref[1], preferred_element_type=jnp.float32)
       + jnp.dot(h, w4_ref[2], preferred_element_type=jnp.float32)
       + jnp.dot(lh, w4_ref[3], preferred_element_type=jnp.float32)
       + b_ref[...])
  dh = h.shape[1]
  zi, zf, zc, zo = (z[:, 0:dh], z[:, dh:2 * dh], z[:, 2 * dh:3 * dh],
                    z[:, 3 * dh:4 * dh])
  gi = jax.nn.sigmoid(zi + wc_ref[0:1] * cc)
  gf = jax.nn.sigmoid(zf + wc_ref[1:2] * cc)
  cn = gf * cc + gi * jnp.tanh(zc)
  go = jax.nn.sigmoid(zo + wc_ref[2:3] * cn)
  hn = go * jnp.tanh(cn)
  cn_ref[...] = cn
  hn_ref[...] = hn


def _pool_body(h_ref, b_ref, clsw_ref, clsb_ref, out_ref, *, G):
  n = h_ref.shape[0]
  gids = lax.broadcasted_iota(jnp.int32, (n, G), 1)
  onehot = (b_ref[...] == gids).astype(jnp.float32)
  sums = lax.dot_general(onehot, h_ref[...], (((0,), (0,)), ((), ())),
                         preferred_element_type=jnp.float32)
  cnt = jnp.sum(onehot, axis=0)
  pooled = sums / jnp.maximum(cnt, 1.0)[:, None]
  out_ref[...] = (jnp.dot(pooled, clsw_ref[...],
                          preferred_element_type=jnp.float32) + clsb_ref[...])


def kernel(x, edge_index, edge_attr, batch, conv_x_W, conv_x_b, conv_h_W,
           conv_h_b, w_c, b_gate, cls_W, cls_b):
  T, N, D = x.shape
  E = edge_index.shape[1]
  G = 16
  DH = conv_x_W.shape[-1]
  DO = cls_W.shape[1]
  EPU = NW * CB
  EP = ((E + EPU - 1) // EPU) * EPU
  NP = ((N + 127) // 128) * 128 + 8

  epad = EP - E
  row = jnp.pad(edge_index[0], (0, epad))
  col = jnp.pad(edge_index[1], (0, epad))
  w_flat = jnp.pad(edge_attr, ((0, 0), (0, epad))).reshape(-1)

  degp = _make_deg(NP, EP, T)(row, w_flat)

  dis = pl.pallas_call(
      _prep_body,
      out_shape=jax.ShapeDtypeStruct((T, NP), jnp.float32),
  )(degp.reshape(NW, T, NP))
  dis_flat = dis.reshape(-1)

  xT = jnp.pad(x.transpose(0, 2, 1), ((0, 0), (0, 0), (0, NP - N)))
  lxT = _make_spmm(NP, EP, D, T)(xT.reshape(-1), row, col, w_flat, dis_flat)
  lx = lxT.reshape(T, D, NP).transpose(0, 2, 1)
  disC = dis.T

  spmm1 = _make_spmm(NP, EP, D, 1)

  wx0 = jnp.transpose(conv_x_W[:, 0], (1, 0, 2)).reshape(D, 4 * DH)
  wx1 = jnp.transpose(conv_x_W[:, 1], (1, 0, 2)).reshape(D, 4 * DH)
  wh0 = jnp.transpose(conv_h_W[:, 0], (1, 0, 2)).reshape(DH, 4 * DH)
  wh1 = jnp.transpose(conv_h_W[:, 1], (1, 0, 2)).reshape(DH, 4 * DH)
  w4 = jnp.stack([wx0, wx1, wh0, wh1])
  bias = (conv_x_b + conv_h_b + b_gate).reshape(1, 4 * DH)

  R = 2000
  step_call = pl.pallas_call(
      _step_body,
      grid=(N // R,),
      in_specs=[
          pl.BlockSpec((R, D), lambda i: (i, 0)),
          pl.BlockSpec((R, D), lambda i: (i, 0)),
          pl.BlockSpec((R, DH), lambda i: (i, 0)),
          pl.BlockSpec((R, DH), lambda i: (i, 0)),
          pl.BlockSpec((R, DH), lambda i: (i, 0)),
          pl.BlockSpec((R, 1), lambda i: (i, 0)),
          pl.BlockSpec((4, D, 4 * DH), lambda i: (0, 0, 0)),
          pl.BlockSpec((1, 4 * DH), lambda i: (0, 0)),
          pl.BlockSpec((3, DH), lambda i: (0, 0)),
      ],
      out_specs=[
          pl.BlockSpec((R, DH), lambda i: (i, 0)),
          pl.BlockSpec((R, DH), lambda i: (i, 0)),
      ],
      out_shape=[
          jax.ShapeDtypeStruct((N, DH), jnp.float32),
          jax.ShapeDtypeStruct((N, DH), jnp.float32),
      ],
  )

  H = jnp.zeros((N, DH), jnp.float32)
  C = jnp.zeros((N, DH), jnp.float32)
  zero_lh = jnp.zeros((NP, DH), jnp.float32)
  for t in range(T):
    if t == 0:
      lh = zero_lh
    else:
      ht = jnp.pad(H.T, ((0, 0), (0, NP - N)))
      lh = spmm1(ht.reshape(-1), row, col, w_flat[t * EP:(t + 1) * EP],
                 dis_flat[t * NP:(t + 1) * NP]).reshape(DH, NP).T
    H, C = step_call(x[t], lx[t], lh, H, C, disC[:, t:t + 1], w4, bias, w_c)

  out = pl.pallas_call(
      functools.partial(_pool_body, G=G),
      out_shape=jax.ShapeDtypeStruct((G, DO), jnp.float32),
  )(H, batch.reshape(N, 1), cls_W, cls_b.reshape(1, DO))
  return out
